# Initial kernel scaffold; baseline (speedup 1.0000x reference)
import jax
import jax.numpy as jnp
from jax.experimental import pallas as pl


def kernel(points, temporal_features, spatial_features, W_lin, bn_gamma, bn_beta, batch_size, spatial_features_stride):
    raise NotImplementedError("write your pallas kernel here")


# speedup vs baseline: 5.9035x; 5.9035x over previous
import functools

import jax
import jax.numpy as jnp
from jax import lax
from jax.experimental import pallas as pl
from jax.experimental.pallas import tpu as pltpu
from jax.experimental.pallas import tpu_sc as plsc

_N = 200000
_C = 256
_H = 128
_W = 128
_VOXEL_X = 0.1
_VOXEL_Y = 0.1
_PCR_X = -51.2
_PCR_Y = -51.2

_NC = 2
_NS = 16
_NW = _NC * _NS
_LANES = 16

_CHUNK = 32
_PTS_PER_W = 6400
_N_PAD = 204800


def _floor_i32(v):
    t = v.astype(jnp.int32)
    return jnp.where(t.astype(jnp.float32) > v, t - 1, t)


def _sc_gather(x_hbm, y_hbm, tt_hbm, ts_hbm,
               oat, obt, oct_, odt, oas, obs, ocs, ods,
               xv, yv, idxv, gt, gs, sem_t, sem_s):
    wid = lax.axis_index("s") * _NC + lax.axis_index("c")
    base = wid * _PTS_PER_W
    nchunks = jnp.where(wid < _NW - 1, _PTS_PER_W // _CHUNK, 50)

    pltpu.sync_copy(x_hbm.at[pl.ds(base, _PTS_PER_W)], xv)
    pltpu.sync_copy(y_hbm.at[pl.ds(base, _PTS_PER_W)], yv)

    def chunk_body(ci, carry):
        for g in range(_CHUNK // _LANES):
            off = ci * _CHUNK + g * _LANES
            xg = xv[pl.ds(off, _LANES)]
            yg = yv[pl.ds(off, _LANES)]
            x0 = _floor_i32(xg)
            y0 = _floor_i32(yg)
            x0c = jnp.clip(x0, 0, _W - 1)
            x1c = jnp.clip(x0 + 1, 0, _W - 1)
            y0c = jnp.clip(y0, 0, _H - 1)
            y1c = jnp.clip(y0 + 1, 0, _H - 1)
            idxv[pl.ds(0 * _CHUNK + g * _LANES, _LANES)] = y0c * _W + x0c
            idxv[pl.ds(1 * _CHUNK + g * _LANES, _LANES)] = y1c * _W + x0c
            idxv[pl.ds(2 * _CHUNK + g * _LANES, _LANES)] = y0c * _W + x1c
            idxv[pl.ds(3 * _CHUNK + g * _LANES, _LANES)] = y1c * _W + x1c

        cp_t = pltpu.async_copy(tt_hbm.at[idxv], gt, sem_t)
        cp_s = pltpu.async_copy(ts_hbm.at[idxv], gs, sem_s)
        cp_t.wait()
        cp_s.wait()
        dst = pl.ds(base + ci * _CHUNK, _CHUNK)
        pltpu.sync_copy(gt.at[pl.ds(0 * _CHUNK, _CHUNK)], oat.at[dst])
        pltpu.sync_copy(gt.at[pl.ds(1 * _CHUNK, _CHUNK)], obt.at[dst])
        pltpu.sync_copy(gt.at[pl.ds(2 * _CHUNK, _CHUNK)], oct_.at[dst])
        pltpu.sync_copy(gt.at[pl.ds(3 * _CHUNK, _CHUNK)], odt.at[dst])
        pltpu.sync_copy(gs.at[pl.ds(0 * _CHUNK, _CHUNK)], oas.at[dst])
        pltpu.sync_copy(gs.at[pl.ds(1 * _CHUNK, _CHUNK)], obs.at[dst])
        pltpu.sync_copy(gs.at[pl.ds(2 * _CHUNK, _CHUNK)], ocs.at[dst])
        pltpu.sync_copy(gs.at[pl.ds(3 * _CHUNK, _CHUNK)], ods.at[dst])
        return carry

    lax.fori_loop(0, nchunks, chunk_body, 0)


def kernel(points, temporal_features, spatial_features, W_lin, bn_gamma, bn_beta, batch_size, spatial_features_stride):
    x = (points[:, 1] - _PCR_X) / _VOXEL_X / spatial_features_stride
    y = (points[:, 2] - _PCR_Y) / _VOXEL_Y / spatial_features_stride
    xp = jnp.pad(x, (0, _N_PAD - _N))
    yp = jnp.pad(y, (0, _N_PAD - _N))
    tt = temporal_features[0].reshape(_C, _H * _W).T
    ts = spatial_features[0].reshape(_C, _H * _W).T

    out1 = jax.ShapeDtypeStruct((_N, _C), jnp.float32)
    sc_call = functools.partial(
        pl.kernel,
        out_type=[out1] * 8,
        mesh=plsc.VectorSubcoreMesh(core_axis_name="c", subcore_axis_name="s"),
        scratch_types=[
            pltpu.VMEM((_PTS_PER_W,), jnp.float32),
            pltpu.VMEM((_PTS_PER_W,), jnp.float32),
            pltpu.VMEM((4 * _CHUNK,), jnp.int32),
            pltpu.VMEM((4 * _CHUNK, _C), jnp.float32),
            pltpu.VMEM((4 * _CHUNK, _C), jnp.float32),
            pltpu.SemaphoreType.DMA,
            pltpu.SemaphoreType.DMA,
        ],
    )
    Iat, Ibt, Ict, Idt, Ias, Ibs, Ics, Ids = sc_call(_sc_gather)(xp, yp, tt, ts)

    x0 = jnp.floor(x).astype(jnp.int32)
    x1 = x0 + 1
    y0 = jnp.floor(y).astype(jnp.int32)
    y1 = y0 + 1
    x0c = jnp.clip(x0, 0, _W - 1)
    x1c = jnp.clip(x1, 0, _W - 1)
    y0c = jnp.clip(y0, 0, _H - 1)
    y1c = jnp.clip(y1, 0, _H - 1)
    wa = (x1c.astype(x.dtype) - x) * (y1c.astype(y.dtype) - y)
    wb = (x1c.astype(x.dtype) - x) * (y - y0c.astype(y.dtype))
    wc = (x - x0c.astype(x.dtype)) * (y1c.astype(y.dtype) - y)
    wd = (x - x0c.astype(x.dtype)) * (y - y0c.astype(y.dtype))
    feats = []
    for Ia, Ib, Ic, Id in ((Iat, Ibt, Ict, Idt), (Ias, Ibs, Ics, Ids)):
        feats.append(Ia * wa[:, None] + Ib * wb[:, None]
                     + Ic * wc[:, None] + Id * wd[:, None])
    pf = jnp.concatenate(feats, axis=-1)

    h = pf @ W_lin
    mean = jnp.mean(h, axis=0)
    var = jnp.var(h, axis=0)
    h = (h - mean) / jnp.sqrt(var + 1e-5) * bn_gamma + bn_beta
    return jax.nn.relu(h)
